# baseline (speedup 1.0000x reference)
import functools
import math

import jax
import jax.numpy as jnp
from jax import lax
from jax.experimental import pallas as pl
from jax.experimental.pallas import tpu as pltpu
from jax.experimental.pallas import tpu_sc as plsc

_CUBIC = -0.014
_QUARTIC = 5.6e-05
_PENTIC = -7e-07
_SEXTIC = 2.2e-08

_NC = 2
_NS = 16
_NW = _NC * _NS
_L = 16
_SUB = 256

_PI = math.pi
_A0 = 1.5707288
_A1 = -0.2121144
_A2 = 0.0742610
_A3 = -0.0187293


def _rsqrt(x):
    i = plsc.bitcast(x, jnp.int32)
    i = jnp.int32(0x5F3759DF) - lax.shift_right_logical(i, 1)
    y = plsc.bitcast(i, jnp.float32)
    for _ in range(3):
        y = y * (1.5 - 0.5 * x * y * y)
    return y


def _arccos(x):
    a = jnp.abs(x)
    p = ((_A3 * a + _A2) * a + _A1) * a + _A0
    u = 1.0 - a
    s = u * _rsqrt(jnp.maximum(u, 1e-30))
    r = s * p
    return jnp.where(x >= 0.0, r, _PI - r)


def _make_kernel(nsub, nwords):
    mesh = plsc.VectorSubcoreMesh(core_axis_name="c", subcore_axis_name="s")
    chunk = nsub * _SUB
    idx_t = pltpu.VMEM((_SUB,), jnp.int32)
    comp_t = pltpu.VMEM((_SUB,), jnp.float32)

    @functools.partial(
        pl.kernel,
        out_type=jax.ShapeDtypeStruct((_NW, _L), jnp.float32),
        mesh=mesh,
        compiler_params=pltpu.CompilerParams(needs_layout_passes=False),
        scratch_types=[
            pltpu.VMEM_SHARED((nwords,), jnp.float32),
            [pltpu.VMEM((nsub, _SUB), jnp.int32)] * 3,
            pltpu.VMEM((chunk,), jnp.float32),
            pltpu.VMEM((chunk,), jnp.float32),
            [idx_t] * 9,
            [comp_t] * 9,
            pltpu.VMEM((_L,), jnp.float32),
            [pltpu.SemaphoreType.DMA] * 2,
        ],
    )
    def angle_energy(coords_hbm, rows_hbm, t0_hbm, kk_hbm, out_hbm,
                     csh_v, rows_v, t0_v, kk_v,
                     idx_a, buf_a, acc_v, sems):
        s_id = lax.axis_index("s")
        w = s_id * _NC + lax.axis_index("c")

        @pl.when(s_id == 0)
        def _():
            pltpu.sync_copy(coords_hbm, csh_v)

        for p in range(3):
            pltpu.sync_copy(rows_hbm[p].at[w], rows_v[p])
        pltpu.sync_copy(t0_hbm.at[pl.ds(w * chunk, chunk)], t0_v)
        pltpu.sync_copy(kk_hbm.at[pl.ds(w * chunk, chunk)], kk_v)
        plsc.subcore_barrier()

        lanes = lax.iota(jnp.int32, _L)

        def build(si, idxb):
            for g in range(_SUB // _L):
                sl = pl.ds(g * _L, _L)
                bi = rows_v[0][si, sl]
                bj = rows_v[1][si, sl]
                bk = rows_v[2][si, sl]
                idxb[0][sl] = bi
                idxb[1][sl] = bi + 1
                idxb[2][sl] = bi + 2
                idxb[3][sl] = bj
                idxb[4][sl] = bj + 1
                idxb[5][sl] = bj + 2
                idxb[6][sl] = bk
                idxb[7][sl] = bk + 1
                idxb[8][sl] = bk + 2

        def fire(idxb, buf, sem):
            for n in range(9):
                pltpu.async_copy(csh_v.at[idxb[n]], buf[n], sem)

        def drain(idxb, buf, sem):
            for n in range(9):
                pltpu.make_async_copy(csh_v.at[idxb[n]], buf[n], sem).wait()

        def compute(si, buf, acc):
            for g in range(_SUB // _L):
                sl = pl.ds(g * _L, _L)
                xi, yi, zi = buf[0][sl], buf[1][sl], buf[2][sl]
                xj, yj, zj = buf[3][sl], buf[4][sl], buf[5][sl]
                xk, yk, zk = buf[6][sl], buf[7][sl], buf[8][sl]
                v1x = xi - xj
                v1y = yi - yj
                v1z = zi - zj
                v2x = xk - xj
                v2y = yk - yj
                v2z = zk - zj
                dot = v1x * v2x + v1y * v2y + v1z * v2z
                m1 = v1x * v1x + v1y * v1y + v1z * v1z
                m2 = v2x * v2x + v2y * v2y + v2z * v2z
                cos = dot * _rsqrt(jnp.maximum(m1 * m2, 1e-30))
                cos = jnp.minimum(jnp.maximum(cos, -1.0), 1.0)
                theta = _arccos(cos)
                base = si * _SUB + g * _L
                t0 = t0_v[pl.ds(base, _L)]
                kk = kk_v[pl.ds(base, _L)]
                dt = theta - t0
                poly = 1.0 + dt * (_CUBIC + dt * (_QUARTIC + dt * (_PENTIC + dt * _SEXTIC)))
                acc = acc + kk * (dt * dt) * poly
            return acc

        def sub(si, acc):
            build(si, idx_a)
            fire(idx_a, buf_a, sems[0])
            drain(idx_a, buf_a, sems[0])
            return compute(si, buf_a, acc)

        acc = lax.fori_loop(0, nsub, sub, jnp.zeros((_L,), jnp.float32))
        acc_v[...] = acc
        pltpu.sync_copy(acc_v, out_hbm.at[w])

    return angle_energy


def kernel(coords, angles, theta0, k):
    m = angles.shape[0]
    n = coords.shape[0]
    group = _NW * _SUB
    nsub = -(-m // group)
    mp = nsub * group
    pad = mp - m
    idx = jnp.pad(angles.astype(jnp.int32), ((0, pad), (0, 0)))
    shape3 = (_NW, nsub, _SUB)
    rows3 = [(idx[:, p] * 3).reshape(shape3) for p in range(3)]
    t0 = jnp.pad(theta0.astype(jnp.float32), (0, pad))
    kk = jnp.pad(k.astype(jnp.float32), (0, pad))
    coords_flat = coords.astype(jnp.float32).reshape(-1)
    partials = _make_kernel(nsub, 3 * n)(coords_flat, rows3, t0, kk)
    return jnp.sum(partials)


# speedup vs baseline: 1.8945x; 1.8945x over previous
import functools
import math

import jax
import jax.numpy as jnp
from jax import lax
from jax.experimental import pallas as pl
from jax.experimental.pallas import tpu as pltpu
from jax.experimental.pallas import tpu_sc as plsc

_CUBIC = -0.014
_QUARTIC = 5.6e-05
_PENTIC = -7e-07
_SEXTIC = 2.2e-08

_NC = 2
_NS = 16
_NW = _NC * _NS
_L = 16
_SUB = 128

_PI = math.pi
_A0 = 1.5707288
_A1 = -0.2121144
_A2 = 0.0742610
_A3 = -0.0187293


def _rsqrt(x):
    i = plsc.bitcast(x, jnp.int32)
    i = jnp.int32(0x5F3759DF) - lax.shift_right_logical(i, 1)
    y = plsc.bitcast(i, jnp.float32)
    for _ in range(3):
        y = y * (1.5 - 0.5 * x * y * y)
    return y


def _arccos(x):
    a = jnp.abs(x)
    p = ((_A3 * a + _A2) * a + _A1) * a + _A0
    u = 1.0 - a
    s = u * _rsqrt(jnp.maximum(u, 1e-30))
    r = s * p
    return jnp.where(x >= 0.0, r, _PI - r)


def _make_kernel(nsub, nwords):
    mesh = plsc.VectorSubcoreMesh(core_axis_name="c", subcore_axis_name="s")
    chunk = nsub * _SUB
    idx_t = pltpu.VMEM((_SUB,), jnp.int32)
    comp_t = pltpu.VMEM((_SUB,), jnp.float32)

    @functools.partial(
        pl.kernel,
        out_type=jax.ShapeDtypeStruct((_NW, _L), jnp.float32),
        mesh=mesh,
        compiler_params=pltpu.CompilerParams(needs_layout_passes=False),
        scratch_types=[
            [pltpu.VMEM_SHARED((nwords,), jnp.float32)] * 3,
            [pltpu.VMEM((nsub, _SUB), jnp.int32)] * 3,
            pltpu.VMEM((chunk,), jnp.float32),
            pltpu.VMEM((chunk,), jnp.float32),
            [idx_t] * 3,
            [comp_t] * 9,
            pltpu.VMEM((_L,), jnp.float32),
            [pltpu.SemaphoreType.DMA] * 2,
        ],
    )
    def angle_energy(coords_hbm, rows_hbm, t0_hbm, kk_hbm, out_hbm,
                     csh_v, rows_v, t0_v, kk_v,
                     idx_a, buf_a, acc_v, sems):
        s_id = lax.axis_index("s")
        w = s_id * _NC + lax.axis_index("c")

        for d in range(3):
            @pl.when(s_id == d)
            def _(d=d):
                pltpu.sync_copy(coords_hbm[d], csh_v[d])

        for p in range(3):
            pltpu.sync_copy(rows_hbm[p].at[w], rows_v[p])
        pltpu.sync_copy(t0_hbm.at[pl.ds(w * chunk, chunk)], t0_v)
        pltpu.sync_copy(kk_hbm.at[pl.ds(w * chunk, chunk)], kk_v)
        plsc.subcore_barrier()

        lanes = lax.iota(jnp.int32, _L)

        def build(si, idxb):
            for g in range(_SUB // _L):
                sl = pl.ds(g * _L, _L)
                idxb[0][sl] = rows_v[0][si, sl]
                idxb[1][sl] = rows_v[1][si, sl]
                idxb[2][sl] = rows_v[2][si, sl]

        def fire(idxb, buf, sem):
            for v in range(3):
                for d in range(3):
                    pltpu.async_copy(csh_v[d].at[idxb[v]], buf[3 * v + d], sem)

        def drain(idxb, buf, sem):
            for v in range(3):
                for d in range(3):
                    pltpu.make_async_copy(
                        csh_v[d].at[idxb[v]], buf[3 * v + d], sem).wait()

        def compute(si, buf, acc):
            for g in range(_SUB // _L):
                sl = pl.ds(g * _L, _L)
                xi, yi, zi = buf[0][sl], buf[1][sl], buf[2][sl]
                xj, yj, zj = buf[3][sl], buf[4][sl], buf[5][sl]
                xk, yk, zk = buf[6][sl], buf[7][sl], buf[8][sl]
                v1x = xi - xj
                v1y = yi - yj
                v1z = zi - zj
                v2x = xk - xj
                v2y = yk - yj
                v2z = zk - zj
                dot = v1x * v2x + v1y * v2y + v1z * v2z
                m1 = v1x * v1x + v1y * v1y + v1z * v1z
                m2 = v2x * v2x + v2y * v2y + v2z * v2z
                cos = dot * _rsqrt(jnp.maximum(m1 * m2, 1e-30))
                cos = jnp.minimum(jnp.maximum(cos, -1.0), 1.0)
                theta = _arccos(cos)
                base = si * _SUB + g * _L
                t0 = t0_v[pl.ds(base, _L)]
                kk = kk_v[pl.ds(base, _L)]
                dt = theta - t0
                poly = 1.0 + dt * (_CUBIC + dt * (_QUARTIC + dt * (_PENTIC + dt * _SEXTIC)))
                acc = acc + kk * (dt * dt) * poly
            return acc

        def sub(si, acc):
            build(si, idx_a)
            fire(idx_a, buf_a, sems[0])
            drain(idx_a, buf_a, sems[0])
            return compute(si, buf_a, acc)

        acc = lax.fori_loop(0, nsub, sub, jnp.zeros((_L,), jnp.float32))
        acc_v[...] = acc
        pltpu.sync_copy(acc_v, out_hbm.at[w])

    return angle_energy


def kernel(coords, angles, theta0, k):
    m = angles.shape[0]
    n = coords.shape[0]
    group = _NW * _SUB
    nsub = -(-m // group)
    mp = nsub * group
    pad = mp - m
    idx = jnp.pad(angles.astype(jnp.int32), ((0, pad), (0, 0)))
    shape3 = (_NW, nsub, _SUB)
    rows3 = [idx[:, p].reshape(shape3) for p in range(3)]
    t0 = jnp.pad(theta0.astype(jnp.float32), (0, pad))
    kk = jnp.pad(k.astype(jnp.float32), (0, pad))
    cf = coords.astype(jnp.float32)
    xyz = [cf[:, d] for d in range(3)]
    partials = _make_kernel(nsub, n)(xyz, rows3, t0, kk)
    return jnp.sum(partials)
